# baseline (speedup 1.0000x reference)
import functools

import jax
import jax.numpy as jnp
from jax import lax
from jax.experimental import pallas as pl
from jax.experimental.pallas import tpu as pltpu
from jax.experimental.pallas import tpu_sc as plsc

RATIO = 16
LANES = 16
NUM_WORKERS = 32
BLK_ROWS = 1024


def _build_sc_mask_kernel(rows):
    wpw = rows // NUM_WORKERS
    mesh = plsc.VectorSubcoreMesh(core_axis_name="c", subcore_axis_name="s")

    @functools.partial(
        pl.kernel,
        mesh=mesh,
        out_type=[jax.ShapeDtypeStruct((rows,), jnp.int32)],
        scratch_types=[
            pltpu.VMEM((RATIO * wpw,), jnp.int32),
            pltpu.VMEM((wpw,), jnp.int32),
            pltpu.SemaphoreType.DMA,
        ],
    )
    def sc_mask(mask_hbm, gmask_hbm, mask_v, gout_v, sem_mask):
        wid = lax.axis_index("s") * 2 + lax.axis_index("c")
        base = wid * wpw
        cp = pltpu.async_copy(
            mask_hbm.at[pl.ds(wid * RATIO * wpw, RATIO * wpw)], mask_v,
            sem_mask)
        cp.wait()
        for g in range(wpw // LANES):
            acc = mask_v[pl.ds(g * LANES, LANES)]
            for k in range(1, RATIO):
                acc = jnp.minimum(
                    acc, mask_v[pl.ds(k * wpw + g * LANES, LANES)])
            gout_v[pl.ds(g * LANES, LANES)] = acc
        pltpu.sync_copy(gout_v, gmask_hbm.at[pl.ds(base, wpw)])

    return sc_mask


def _tc_fill(embeds, rows, num_global, hidden):
    def body(emb_ref, out_ref):
        i = pl.program_id(0)
        row = i * BLK_ROWS + lax.broadcasted_iota(
            jnp.int32, (BLK_ROWS, 1), 0)
        is_boundary = (row % num_global) == 0
        out_ref[...] = jnp.where(
            is_boundary, emb_ref[1][None, :], emb_ref[0][None, :])

    return pl.pallas_call(
        body,
        grid=(rows // BLK_ROWS,),
        in_specs=[pl.BlockSpec((2, hidden), lambda i: (0, 0))],
        out_specs=pl.BlockSpec((BLK_ROWS, hidden), lambda i: (i, 0)),
        out_shape=jax.ShapeDtypeStruct((rows, hidden), jnp.float32),
    )(embeds)


def kernel(token_ids, padding_mask, embeds):
    batch, seq_len = token_ids.shape
    hidden = embeds.shape[1]
    num_global = seq_len // RATIO
    rows = batch * num_global
    wpw = rows // NUM_WORKERS
    mask_t = (
        padding_mask.astype(jnp.int32)
        .reshape(NUM_WORKERS, wpw, RATIO)
        .transpose(0, 2, 1)
        .reshape(batch * seq_len)
    )
    (gmask,) = _build_sc_mask_kernel(rows)(mask_t)
    out_flat = _tc_fill(embeds[:2], rows, num_global, hidden)
    out = out_flat.reshape(batch, num_global, hidden)
    gmask = gmask.reshape(batch, num_global).astype(jnp.bool_)
    return out, gmask


# speedup vs baseline: 1.0339x; 1.0339x over previous
import functools

import jax
import jax.numpy as jnp
from jax import lax
from jax.experimental import pallas as pl
from jax.experimental.pallas import tpu as pltpu
from jax.experimental.pallas import tpu_sc as plsc

RATIO = 16
LANES = 16
NUM_WORKERS = 32
BLK_ROWS = 512


def _build_sc_mask_kernel(rows):
    wpw = rows // NUM_WORKERS
    mesh = plsc.VectorSubcoreMesh(core_axis_name="c", subcore_axis_name="s")

    @functools.partial(
        pl.kernel,
        mesh=mesh,
        out_type=[jax.ShapeDtypeStruct((rows,), jnp.int32)],
        scratch_types=[
            pltpu.VMEM((RATIO * wpw,), jnp.int32),
            pltpu.VMEM((wpw,), jnp.int32),
            pltpu.SemaphoreType.DMA,
        ],
    )
    def sc_mask(mask_hbm, gmask_hbm, mask_v, gout_v, sem_mask):
        wid = lax.axis_index("s") * 2 + lax.axis_index("c")
        base = wid * wpw
        cp = pltpu.async_copy(
            mask_hbm.at[pl.ds(wid * RATIO * wpw, RATIO * wpw)], mask_v,
            sem_mask)
        cp.wait()
        for g in range(wpw // LANES):
            acc = mask_v[pl.ds(g * LANES, LANES)]
            for k in range(1, RATIO):
                acc = jnp.minimum(
                    acc, mask_v[pl.ds(k * wpw + g * LANES, LANES)])
            gout_v[pl.ds(g * LANES, LANES)] = acc
        pltpu.sync_copy(gout_v, gmask_hbm.at[pl.ds(base, wpw)])

    return sc_mask


def _tc_fill(embeds, rows, num_global, hidden):
    def body(emb_ref, out_ref):
        i = pl.program_id(0)
        row = i * BLK_ROWS + lax.broadcasted_iota(
            jnp.int32, (BLK_ROWS, 1), 0)
        is_boundary = (row % num_global) == 0
        out_ref[...] = jnp.where(
            is_boundary, emb_ref[1][None, :], emb_ref[0][None, :])

    return pl.pallas_call(
        body,
        grid=(rows // BLK_ROWS,),
        in_specs=[pl.BlockSpec((8, hidden), lambda i: (0, 0))],
        out_specs=pl.BlockSpec((BLK_ROWS, hidden), lambda i: (i, 0)),
        out_shape=jax.ShapeDtypeStruct((rows, hidden), jnp.float32),
    )(embeds)


def kernel(token_ids, padding_mask, embeds):
    batch, seq_len = token_ids.shape
    hidden = embeds.shape[1]
    num_global = seq_len // RATIO
    rows = batch * num_global
    wpw = rows // NUM_WORKERS
    mask_t = (
        padding_mask.astype(jnp.int32)
        .reshape(NUM_WORKERS, wpw, RATIO)
        .transpose(0, 2, 1)
        .reshape(batch * seq_len)
    )
    (gmask,) = _build_sc_mask_kernel(rows)(mask_t)
    out_flat = _tc_fill(embeds, rows, num_global, hidden)
    out = out_flat.reshape(batch, num_global, hidden)
    gmask = gmask.reshape(batch, num_global).astype(jnp.bool_)
    return out, gmask
